# baseline (speedup 1.0000x reference)
import functools

import jax
import jax.numpy as jnp
from jax import lax
from jax.experimental import pallas as pl
from jax.experimental.pallas import tpu as pltpu
from jax.experimental.pallas import tpu_sc as plsc

_N = 16384
_ACTION_DIM = 7
_EXPOSURE_THRESHOLD = 0.9

_NC, _NS, _L = 2, 16, 16
_NW = _NC * _NS
_RPW = _N // _NW
_CHUNKS = _RPW // _L

_mesh = plsc.VectorSubcoreMesh(core_axis_name="c", subcore_axis_name="s")


@functools.partial(
    pl.kernel,
    mesh=_mesh,
    out_type=jax.ShapeDtypeStruct((_ACTION_DIM * _N,), jnp.int32),
    scratch_types=[
        pltpu.VMEM((_RPW,), jnp.float32),
        pltpu.VMEM((_RPW,), jnp.float32),
        pltpu.VMEM((_RPW,), jnp.int32),
        pltpu.VMEM((_RPW,), jnp.int32),
        pltpu.VMEM((_RPW,), jnp.int32),
        pltpu.VMEM((_RPW,), jnp.int32),
        pltpu.SemaphoreType.DMA,
        pltpu.SemaphoreType.DMA,
    ],
)
def _mask_sc(pos_hbm, expo_hbm, out_hbm, pos_v, expo_v,
             hold_v, buy_v, sell_v, sinc_v, sem_in, sem_out):
    wid = lax.axis_index("s") * _NC + lax.axis_index("c")
    base = wid * _RPW
    cp_pos = pltpu.async_copy(pos_hbm.at[pl.ds(base, _RPW)], pos_v, sem_in)
    cp_expo = pltpu.async_copy(expo_hbm.at[pl.ds(base, _RPW)], expo_v, sem_in)
    cp_pos.wait()
    cp_expo.wait()

    ones = jnp.full((_L,), 1, jnp.int32)
    zeros = jnp.zeros((_L,), jnp.int32)

    for i in range(_CHUNKS):
        sl = pl.ds(i * _L, _L)
        sell = jnp.where(pos_v[sl] > 0.0, ones, zeros)
        not_high = jnp.where(expo_v[sl] >= _EXPOSURE_THRESHOLD, zeros, ones)
        hold_v[sl] = ones
        buy_v[sl] = (ones - sell) * not_high
        sell_v[sl] = sell
        sinc_v[sl] = sell * not_high

    cps = [
        pltpu.async_copy(hold_v, out_hbm.at[pl.ds(0 * _N + base, _RPW)], sem_out),
        pltpu.async_copy(buy_v, out_hbm.at[pl.ds(1 * _N + base, _RPW)], sem_out),
        pltpu.async_copy(buy_v, out_hbm.at[pl.ds(2 * _N + base, _RPW)], sem_out),
        pltpu.async_copy(buy_v, out_hbm.at[pl.ds(3 * _N + base, _RPW)], sem_out),
        pltpu.async_copy(sell_v, out_hbm.at[pl.ds(4 * _N + base, _RPW)], sem_out),
        pltpu.async_copy(sell_v, out_hbm.at[pl.ds(5 * _N + base, _RPW)], sem_out),
        pltpu.async_copy(sinc_v, out_hbm.at[pl.ds(6 * _N + base, _RPW)], sem_out),
    ]
    for cp in cps:
        cp.wait()


def kernel(position, portfolio):
    pos_col = position.astype(jnp.float32)[:, 1]
    expo_col = portfolio.astype(jnp.float32)[:, 1]
    out = _mask_sc(pos_col, expo_col)
    return out.reshape(_ACTION_DIM, _N).T != 0


# speedup vs baseline: 1.0392x; 1.0392x over previous
import functools

import jax
import jax.numpy as jnp
from jax import lax
from jax.experimental import pallas as pl
from jax.experimental.pallas import tpu as pltpu
from jax.experimental.pallas import tpu_sc as plsc

_N = 16384
_ACTION_DIM = 7
_EXPOSURE_THRESHOLD = 0.9

_NC, _NS, _L = 1, 16, 16
_NW = _NC * _NS
_RPW = _N // _NW
_CHUNKS = _RPW // _L

_mesh = plsc.VectorSubcoreMesh(core_axis_name="c", subcore_axis_name="s",
                               num_cores=1)


@functools.partial(
    pl.kernel,
    mesh=_mesh,
    out_type=jax.ShapeDtypeStruct((_ACTION_DIM * _N,), jnp.int32),
    scratch_types=[
        pltpu.VMEM((_RPW,), jnp.float32),
        pltpu.VMEM((_RPW,), jnp.float32),
        pltpu.VMEM((_RPW,), jnp.int32),
        pltpu.VMEM((_RPW,), jnp.int32),
        pltpu.VMEM((_RPW,), jnp.int32),
        pltpu.VMEM((_RPW,), jnp.int32),
        pltpu.SemaphoreType.DMA,
        pltpu.SemaphoreType.DMA,
    ],
)
def _mask_sc(pos_hbm, expo_hbm, out_hbm, pos_v, expo_v,
             hold_v, buy_v, sell_v, sinc_v, sem_in, sem_out):
    wid = lax.axis_index("s") * _NC + lax.axis_index("c")
    base = wid * _RPW
    cp_pos = pltpu.async_copy(pos_hbm.at[pl.ds(base, _RPW)], pos_v, sem_in)
    cp_expo = pltpu.async_copy(expo_hbm.at[pl.ds(base, _RPW)], expo_v, sem_in)
    cp_pos.wait()
    cp_expo.wait()

    ones = jnp.full((_L,), 1, jnp.int32)
    zeros = jnp.zeros((_L,), jnp.int32)

    for i in range(_CHUNKS):
        sl = pl.ds(i * _L, _L)
        sell = jnp.where(pos_v[sl] > 0.0, ones, zeros)
        not_high = jnp.where(expo_v[sl] >= _EXPOSURE_THRESHOLD, zeros, ones)
        hold_v[sl] = ones
        buy_v[sl] = (ones - sell) * not_high
        sell_v[sl] = sell
        sinc_v[sl] = sell * not_high

    cps = [
        pltpu.async_copy(hold_v, out_hbm.at[pl.ds(0 * _N + base, _RPW)], sem_out),
        pltpu.async_copy(buy_v, out_hbm.at[pl.ds(1 * _N + base, _RPW)], sem_out),
        pltpu.async_copy(buy_v, out_hbm.at[pl.ds(2 * _N + base, _RPW)], sem_out),
        pltpu.async_copy(buy_v, out_hbm.at[pl.ds(3 * _N + base, _RPW)], sem_out),
        pltpu.async_copy(sell_v, out_hbm.at[pl.ds(4 * _N + base, _RPW)], sem_out),
        pltpu.async_copy(sell_v, out_hbm.at[pl.ds(5 * _N + base, _RPW)], sem_out),
        pltpu.async_copy(sinc_v, out_hbm.at[pl.ds(6 * _N + base, _RPW)], sem_out),
    ]
    for cp in cps:
        cp.wait()


def kernel(position, portfolio):
    pos_col = position.astype(jnp.float32)[:, 1]
    expo_col = portfolio.astype(jnp.float32)[:, 1]
    out = _mask_sc(pos_col, expo_col)
    return out.reshape(_ACTION_DIM, _N).T != 0
